# baseline (speedup 1.0000x reference)
import jax
import jax.numpy as jnp
from jax import lax
from jax.experimental import pallas as pl

_N_CLASSES = 1000


def _onehot_body(idx_ref, out_ref):
    j = pl.program_id(0)
    idx = idx_ref[pl.ds(j, 1), :]
    m, k, n = out_ref.shape
    classes = lax.broadcasted_iota(jnp.int32, (m, k, n), 1)
    out_ref[...] = (idx[:, None, :] == classes).astype(jnp.float32)


def kernel(inputs):
    n, m = inputs.shape
    idx_t = inputs.T
    out_t = pl.pallas_call(
        _onehot_body,
        grid=(m,),
        in_specs=[pl.BlockSpec((m, n), lambda j: (0, 0))],
        out_specs=pl.BlockSpec((1, _N_CLASSES, n), lambda j: (j, 0, 0)),
        out_shape=jax.ShapeDtypeStruct((m, _N_CLASSES, n), jnp.float32),
    )(idx_t)
    return jnp.transpose(out_t, (2, 0, 1))


# speedup vs baseline: 1.0129x; 1.0129x over previous
import jax
import jax.numpy as jnp
from jax import lax
from jax.experimental import pallas as pl
from jax.experimental.pallas import tpu as pltpu

_N_CLASSES = 1000
_BLOCK_K = 200
_KSTEPS = _N_CLASSES // _BLOCK_K
_NBUF = 4


def _onehot_body(idx_ref, out_ref, buf_ref, sem):
    j = pl.program_id(0)
    q = pl.program_id(1)
    i = j * _KSTEPS + q
    n_total = pl.num_programs(0) * _KSTEPS

    idx = idx_ref[pl.ds(j, 1), :]
    classes = lax.broadcasted_iota(
        jnp.int32, (1, _BLOCK_K, idx_ref.shape[1]), 1) + q * _BLOCK_K
    vals = (idx[:, None, :] == classes).astype(jnp.float32)

    for k in range(_NBUF):

        @pl.when(jnp.logical_and(i % _NBUF == k, i >= _NBUF))
        def _wait():
            pltpu.make_async_copy(
                buf_ref.at[k],
                out_ref.at[pl.ds(j, 1), pl.ds(q * _BLOCK_K, _BLOCK_K), :],
                sem.at[k],
            ).wait()

        @pl.when(i % _NBUF == k)
        def _issue():
            buf_ref[k] = vals
            pltpu.make_async_copy(
                buf_ref.at[k],
                out_ref.at[pl.ds(j, 1), pl.ds(q * _BLOCK_K, _BLOCK_K), :],
                sem.at[k],
            ).start()

    @pl.when(i == n_total - 1)
    def _drain():
        for k in range(_NBUF):
            pltpu.make_async_copy(
                buf_ref.at[k],
                out_ref.at[pl.ds(j, 1), pl.ds(q * _BLOCK_K, _BLOCK_K), :],
                sem.at[k],
            ).wait()


def kernel(inputs):
    n, m = inputs.shape
    idx_t = inputs.T
    out_t = pl.pallas_call(
        _onehot_body,
        grid=(m, _KSTEPS),
        in_specs=[pl.BlockSpec((m, n), lambda j, q: (0, 0))],
        out_specs=pl.BlockSpec(memory_space=pl.ANY),
        out_shape=jax.ShapeDtypeStruct((m, _N_CLASSES, n), jnp.float32),
        scratch_shapes=[
            pltpu.VMEM((_NBUF, 1, _BLOCK_K, n), jnp.float32),
            pltpu.SemaphoreType.DMA((_NBUF,)),
        ],
    )(idx_t)
    return jnp.transpose(out_t, (2, 0, 1))
